# Initial kernel scaffold; baseline (speedup 1.0000x reference)
import jax
import jax.numpy as jnp
from jax.experimental import pallas as pl


def kernel(x, cumulative_skipped_flops, step_embeddings_weight):
    raise NotImplementedError("write your pallas kernel here")


# speedup vs baseline: 2.4637x; 2.4637x over previous
import functools

import jax
import jax.numpy as jnp
from jax.experimental import pallas as pl
from jax.experimental.pallas import tpu as pltpu

BATCH = 4
SEQ_LEN = 4096
D_MODEL = 2048
NUM_BUCKETS = 64
MAX_SKIP_LAYERS = 12
_MAX_SKIPPED_FLOPS = float(MAX_SKIP_LAYERS * 12 * D_MODEL * D_MODEL * SEQ_LEN)

_N = BATCH * SEQ_LEN
_BLK = 512


def _tc_body(csf_ref, x_ref, tab_ref, o_ref):
    csf = csf_ref[0, 0, :]
    frac = csf / jnp.float32(_MAX_SKIPPED_FLOPS)
    idx = jnp.floor(frac * NUM_BUCKETS).astype(jnp.int32)
    idx = jnp.clip(idx, 0, NUM_BUCKETS - 1)
    iota = jax.lax.broadcasted_iota(jnp.int32, (_BLK, NUM_BUCKETS), 1)
    onehot = (idx[:, None] == iota).astype(jnp.float32)
    emb = jnp.dot(onehot, tab_ref[...], preferred_element_type=jnp.float32)
    o_ref[...] = x_ref[...] + emb


def kernel(x, cumulative_skipped_flops, step_embeddings_weight):
    x2 = x.reshape(_N, D_MODEL)
    csf3 = cumulative_skipped_flops.reshape(_N // _BLK, 1, _BLK)
    grid = (_N // _BLK,)
    out = pl.pallas_call(
        _tc_body,
        grid=grid,
        in_specs=[
            pl.BlockSpec((1, 1, _BLK), lambda i: (i, 0, 0)),
            pl.BlockSpec((_BLK, D_MODEL), lambda i: (i, 0)),
            pl.BlockSpec((NUM_BUCKETS, D_MODEL), lambda i: (0, 0)),
        ],
        out_specs=pl.BlockSpec((_BLK, D_MODEL), lambda i: (i, 0)),
        out_shape=jax.ShapeDtypeStruct((_N, D_MODEL), jnp.float32),
        compiler_params=pltpu.CompilerParams(
            dimension_semantics=("arbitrary",),
        ),
    )(csf3, x2, step_embeddings_weight)
    return out.reshape(BATCH, SEQ_LEN, D_MODEL)
